# Initial kernel scaffold; baseline (speedup 1.0000x reference)
import jax
import jax.numpy as jnp
from jax.experimental import pallas as pl


def kernel(x, edge_index, W1, b1, W2, b2):
    raise NotImplementedError("write your pallas kernel here")


# speedup vs baseline: 46.5976x; 46.5976x over previous
import functools

import jax
import jax.numpy as jnp
from jax import lax
from jax.experimental import pallas as pl
from jax.experimental.pallas import tpu as pltpu
from jax.experimental.pallas import tpu_sc as plsc

_NC = 2
_NS = 16
_NW = _NC * _NS
_L = 16
_B = 128


def _mesh():
    return plsc.VectorSubcoreMesh(core_axis_name="c", subcore_axis_name="s")

_CP = pltpu.CompilerParams(use_tc_tiling_on_sc=False)


def _sc_degree(dst2d, zeros1, npad, rows_per_tile, sub):
    slab = npad // _NS

    @functools.partial(
        pl.kernel,
        out_type=jax.ShapeDtypeStruct((_NC, npad), jnp.float32),
        mesh=_mesh(),
        compiler_params=_CP,
        scratch_types=[
            pltpu.VMEM((sub, _B), jnp.int32),
            pltpu.VMEM((_B,), jnp.float32),
            pltpu.VMEM_SHARED((npad,), jnp.float32),
            pltpu.SemaphoreType.DMA,
        ],
    )
    def deg_kernel(dst_hbm, z_hbm, out_hbm, idx_v, ones_v, acc_sh, sem):
        c = lax.axis_index("c")
        s = lax.axis_index("s")
        rs = pl.ds(s * slab, slab)
        for i in range(_B // _L):
            ones_v[pl.ds(i * _L, _L)] = jnp.ones((_L,), jnp.float32)
        pltpu.sync_copy(z_hbm.at[rs], acc_sh.at[rs])
        plsc.subcore_barrier()
        base_row = (c * _NS + s) * rows_per_tile

        def body(g, carry):
            r0 = base_row + g * sub
            pltpu.sync_copy(dst_hbm.at[pl.ds(r0, sub)], idx_v)
            cps = [
                pltpu.async_copy(ones_v, acc_sh.at[idx_v.at[j]], sem, add=True)
                for j in range(sub)
            ]
            for cp in cps:
                cp.wait()
            return carry

        lax.fori_loop(0, rows_per_tile // sub, body, 0)
        plsc.subcore_barrier()
        pltpu.sync_copy(acc_sh.at[rs], out_hbm.at[c, rs])

    return deg_kernel(dst2d, zeros1)


def _sc_propagate(src2d, dst2d, table, zeros1, npad, rows_per_tile, nf, sub):
    slab = npad // _NS

    @functools.partial(
        pl.kernel,
        out_type=jax.ShapeDtypeStruct((_NC, nf, npad), jnp.float32),
        mesh=_mesh(),
        compiler_params=_CP,
        scratch_types=[
            pltpu.VMEM((sub, _B), jnp.int32),
            pltpu.VMEM((sub, _B), jnp.int32),
            pltpu.VMEM((sub * nf, _B), jnp.float32),
        ] + [pltpu.VMEM_SHARED((npad,), jnp.float32) for _ in range(2 * nf)] + [
            pltpu.SemaphoreType.DMA,
            pltpu.SemaphoreType.DMA,
        ],
    )
    def prop_kernel(src_hbm, dst_hbm, tab_hbm, z_hbm, out_hbm,
                    sidx_v, didx_v, rows_v, *rest):
        tab_sh = rest[:nf]
        acc_sh = rest[nf:2 * nf]
        gsem, ssem = rest[2 * nf], rest[2 * nf + 1]
        c = lax.axis_index("c")
        s = lax.axis_index("s")
        rs = pl.ds(s * slab, slab)
        for f in range(nf):
            pltpu.sync_copy(tab_hbm.at[f, rs], tab_sh[f].at[rs])
            pltpu.sync_copy(z_hbm.at[rs], acc_sh[f].at[rs])
        plsc.subcore_barrier()
        base_row = (c * _NS + s) * rows_per_tile

        def body(g, carry):
            r0 = base_row + g * sub
            pltpu.sync_copy(src_hbm.at[pl.ds(r0, sub)], sidx_v)
            pltpu.sync_copy(dst_hbm.at[pl.ds(r0, sub)], didx_v)
            gcs = [
                pltpu.async_copy(tab_sh[f].at[sidx_v.at[j]],
                                 rows_v.at[j * nf + f], gsem)
                for j in range(sub) for f in range(nf)
            ]
            for cp in gcs:
                cp.wait()
            scs = [
                pltpu.async_copy(rows_v.at[j * nf + f],
                                 acc_sh[f].at[didx_v.at[j]], ssem, add=True)
                for j in range(sub) for f in range(nf)
            ]
            for cp in scs:
                cp.wait()
            return carry

        lax.fori_loop(0, rows_per_tile // sub, body, 0)
        plsc.subcore_barrier()
        for f in range(nf):
            pltpu.sync_copy(acc_sh[f].at[rs], out_hbm.at[c, f, rs])

    return prop_kernel(src2d, dst2d, table, zeros1)


def _tc_stage1(deg0, deg1, x4, npad, blk):
    def body(d0_ref, d1_ref, x_ref, dinv_ref, xs_ref):
        deg = d0_ref[...] + d1_ref[...] + 1.0
        dinv = lax.rsqrt(deg)
        dinv_ref[...] = dinv
        for f in range(3):
            xs_ref[f, :] = x_ref[:, f] * dinv

    return pl.pallas_call(
        body,
        grid=(npad // blk,),
        in_specs=[
            pl.BlockSpec((blk,), lambda i: (i,)),
            pl.BlockSpec((blk,), lambda i: (i,)),
            pl.BlockSpec((blk, 4), lambda i: (i, 0)),
        ],
        out_specs=[
            pl.BlockSpec((blk,), lambda i: (i,)),
            pl.BlockSpec((3, blk), lambda i: (0, i)),
        ],
        out_shape=[
            jax.ShapeDtypeStruct((npad,), jnp.float32),
            jax.ShapeDtypeStruct((3, npad), jnp.float32),
        ],
    )(deg0, deg1, x4)


def _tc_stage2(a0, a1, x4, dinv, W1, b1, W2, npad, blk):
    def body(a0_ref, a1_ref, x_ref, dinv_ref, w1_ref, b1_ref, w2_ref,
             g_ref, hs_ref):
        dv = dinv_ref[...]
        pre = jnp.broadcast_to(b1_ref[...][None, :], (dv.shape[0], 16))
        for f in range(3):
            p1f = dv * (a0_ref[f, :] + a1_ref[f, :]) + dv * dv * x_ref[:, f]
            pre = pre + p1f[:, None] * w1_ref[f, :][None, :]
        h1 = jnp.maximum(pre, 0.0)
        g = jnp.dot(h1, w2_ref[...], preferred_element_type=jnp.float32)
        g_ref[...] = g
        for f in range(7):
            hs_ref[f, :] = g[:, f] * dv

    return pl.pallas_call(
        body,
        grid=(npad // blk,),
        in_specs=[
            pl.BlockSpec((3, blk), lambda i: (0, i)),
            pl.BlockSpec((3, blk), lambda i: (0, i)),
            pl.BlockSpec((blk, 4), lambda i: (i, 0)),
            pl.BlockSpec((blk,), lambda i: (i,)),
            pl.BlockSpec((3, 16), lambda i: (0, 0)),
            pl.BlockSpec((16,), lambda i: (0,)),
            pl.BlockSpec((16, 8), lambda i: (0, 0)),
        ],
        out_specs=[
            pl.BlockSpec((blk, 8), lambda i: (i, 0)),
            pl.BlockSpec((7, blk), lambda i: (0, i)),
        ],
        out_shape=[
            jax.ShapeDtypeStruct((npad, 8), jnp.float32),
            jax.ShapeDtypeStruct((7, npad), jnp.float32),
        ],
    )(a0, a1, x4, dinv, W1, b1, W2)


def _tc_stage3(a0, a1, g8, dinv, b2p, npad, blk):
    def body(a0_ref, a1_ref, g8_ref, dinv_ref, b2_ref, out_ref):
        dv = dinv_ref[...]
        cols = []
        for f in range(7):
            cols.append((dv * (a0_ref[f, :] + a1_ref[f, :])
                         + dv * dv * g8_ref[:, f] + b2_ref[f])[:, None])
        o = jnp.concatenate(cols, axis=1)
        m = jnp.max(o, axis=1, keepdims=True)
        e = jnp.exp(o - m)
        ssum = jnp.sum(e, axis=1, keepdims=True)
        out_ref[...] = (o - m) - jnp.log(ssum)

    return pl.pallas_call(
        body,
        grid=(npad // blk,),
        in_specs=[
            pl.BlockSpec((7, blk), lambda i: (0, i)),
            pl.BlockSpec((7, blk), lambda i: (0, i)),
            pl.BlockSpec((blk, 8), lambda i: (i, 0)),
            pl.BlockSpec((blk,), lambda i: (i,)),
            pl.BlockSpec((8,), lambda i: (0,)),
        ],
        out_specs=pl.BlockSpec((blk, 7), lambda i: (i, 0)),
        out_shape=jax.ShapeDtypeStruct((npad, 7), jnp.float32),
    )(a0, a1, g8, dinv, b2p)


def kernel(x, edge_index, W1, b1, W2, b2):
    n = x.shape[0]
    e = edge_index.shape[1]
    blk = 2048
    npad = -(-(n + 1) // blk) * blk
    dummy = n
    chunk = 8 * _B
    groups = -(-e // (_NW * chunk))
    ept = groups * chunk
    epad = _NW * ept
    rows_per_tile = ept // _B

    ei = edge_index.astype(jnp.int32)
    pad = jnp.full((epad - e,), dummy, jnp.int32)
    src2d = jnp.concatenate([ei[0], pad]).reshape(epad // _B, _B)
    dst2d = jnp.concatenate([ei[1], pad]).reshape(epad // _B, _B)

    x4 = jnp.zeros((npad, 4), jnp.float32).at[:n, :3].set(x)
    zeros1 = jnp.zeros((npad,), jnp.float32)
    b2p = jnp.zeros((8,), jnp.float32).at[:7].set(b2)
    w2p = jnp.zeros((16, 8), jnp.float32).at[:, :7].set(W2)

    degp = _sc_degree(dst2d, zeros1, npad, rows_per_tile, 8)
    dinv, xs_t = _tc_stage1(degp[0], degp[1], x4, npad, blk)
    aggp = _sc_propagate(src2d, dst2d, xs_t, zeros1, npad, rows_per_tile, 3, 8)
    g8, hs_t = _tc_stage2(aggp[0], aggp[1], x4, dinv, W1, b1, w2p, npad, blk)
    agg2p = _sc_propagate(src2d, dst2d, hs_t, zeros1, npad, rows_per_tile, 7, 4)
    out = _tc_stage3(agg2p[0], agg2p[1], g8, dinv, b2p, npad, blk)
    return out[:n]


# speedup vs baseline: 48.6913x; 1.0449x over previous
import functools

import jax
import jax.numpy as jnp
from jax import lax
from jax.experimental import pallas as pl
from jax.experimental.pallas import tpu as pltpu
from jax.experimental.pallas import tpu_sc as plsc

_NC = 2
_NS = 16
_NW = _NC * _NS
_L = 16
_B = 128


def _mesh():
    return plsc.VectorSubcoreMesh(core_axis_name="c", subcore_axis_name="s")

_CP = pltpu.CompilerParams(use_tc_tiling_on_sc=False)


def _sc_degree(dst2d, zeros1, npad, rows_per_tile, sub):
    slab = npad // _NS

    @functools.partial(
        pl.kernel,
        out_type=jax.ShapeDtypeStruct((_NC, npad), jnp.float32),
        mesh=_mesh(),
        compiler_params=_CP,
        scratch_types=[
            pltpu.VMEM((sub, _B), jnp.int32),
            pltpu.VMEM((_B,), jnp.float32),
            pltpu.VMEM_SHARED((npad,), jnp.float32),
            pltpu.SemaphoreType.DMA,
        ],
    )
    def deg_kernel(dst_hbm, z_hbm, out_hbm, idx_v, ones_v, acc_sh, sem):
        c = lax.axis_index("c")
        s = lax.axis_index("s")
        rs = pl.ds(s * slab, slab)
        for i in range(_B // _L):
            ones_v[pl.ds(i * _L, _L)] = jnp.ones((_L,), jnp.float32)
        pltpu.sync_copy(z_hbm.at[rs], acc_sh.at[rs])
        plsc.subcore_barrier()
        base_row = (c * _NS + s) * rows_per_tile

        def body(g, carry):
            r0 = base_row + g * sub
            pltpu.sync_copy(dst_hbm.at[pl.ds(r0, sub)], idx_v)
            cps = [
                pltpu.async_copy(ones_v, acc_sh.at[idx_v.at[j]], sem, add=True)
                for j in range(sub)
            ]
            for cp in cps:
                cp.wait()
            return carry

        lax.fori_loop(0, rows_per_tile // sub, body, 0)
        plsc.subcore_barrier()
        pltpu.sync_copy(acc_sh.at[rs], out_hbm.at[c, rs])

    return deg_kernel(dst2d, zeros1)


def _sc_propagate(src2d, dst2d, table, zeros1, npad, rows_per_tile, nf, sub):
    slab = npad // _NS

    @functools.partial(
        pl.kernel,
        out_type=jax.ShapeDtypeStruct((_NC, nf, npad), jnp.float32),
        mesh=_mesh(),
        compiler_params=_CP,
        scratch_types=[
            pltpu.VMEM((sub, _B), jnp.int32),
            pltpu.VMEM((sub, _B), jnp.int32),
            pltpu.VMEM((sub * nf, _B), jnp.float32),
        ] + [pltpu.VMEM_SHARED((npad,), jnp.float32) for _ in range(2 * nf)] + [
            pltpu.SemaphoreType.DMA,
            pltpu.SemaphoreType.DMA,
        ],
    )
    def prop_kernel(src_hbm, dst_hbm, tab_hbm, z_hbm, out_hbm,
                    sidx_v, didx_v, rows_v, *rest):
        tab_sh = rest[:nf]
        acc_sh = rest[nf:2 * nf]
        gsem, ssem = rest[2 * nf], rest[2 * nf + 1]
        c = lax.axis_index("c")
        s = lax.axis_index("s")
        rs = pl.ds(s * slab, slab)
        for f in range(nf):
            pltpu.sync_copy(tab_hbm.at[f, rs], tab_sh[f].at[rs])
            pltpu.sync_copy(z_hbm.at[rs], acc_sh[f].at[rs])
        plsc.subcore_barrier()
        base_row = (c * _NS + s) * rows_per_tile

        def body(g, carry):
            r0 = base_row + g * sub
            pltpu.sync_copy(src_hbm.at[pl.ds(r0, sub)], sidx_v)
            pltpu.sync_copy(dst_hbm.at[pl.ds(r0, sub)], didx_v)
            gcs = [
                pltpu.async_copy(tab_sh[f].at[sidx_v.at[j]],
                                 rows_v.at[j * nf + f], gsem)
                for j in range(sub) for f in range(nf)
            ]
            scs = []
            for j in range(sub):
                for f in range(nf):
                    gcs[j * nf + f].wait()
                scs.extend(
                    pltpu.async_copy(rows_v.at[j * nf + f],
                                     acc_sh[f].at[didx_v.at[j]], ssem, add=True)
                    for f in range(nf)
                )
            for cp in scs:
                cp.wait()
            return carry

        lax.fori_loop(0, rows_per_tile // sub, body, 0)
        plsc.subcore_barrier()
        for f in range(nf):
            pltpu.sync_copy(acc_sh[f].at[rs], out_hbm.at[c, f, rs])

    return prop_kernel(src2d, dst2d, table, zeros1)


def _tc_stage1(deg0, deg1, x4, npad, blk, nreal):
    def body(d0_ref, d1_ref, x_ref, dinv_ref, xs_ref):
        deg = d0_ref[...] + d1_ref[...] + 1.0
        row = pl.program_id(0) * blk + lax.iota(jnp.int32, blk)
        dinv = jnp.where(row < nreal, lax.rsqrt(deg), 0.0)
        dinv_ref[...] = dinv
        for f in range(3):
            xs_ref[f, :] = x_ref[:, f] * dinv

    return pl.pallas_call(
        body,
        grid=(npad // blk,),
        in_specs=[
            pl.BlockSpec((blk,), lambda i: (i,)),
            pl.BlockSpec((blk,), lambda i: (i,)),
            pl.BlockSpec((blk, 4), lambda i: (i, 0)),
        ],
        out_specs=[
            pl.BlockSpec((blk,), lambda i: (i,)),
            pl.BlockSpec((3, blk), lambda i: (0, i)),
        ],
        out_shape=[
            jax.ShapeDtypeStruct((npad,), jnp.float32),
            jax.ShapeDtypeStruct((3, npad), jnp.float32),
        ],
    )(deg0, deg1, x4)


def _tc_stage2(a0, a1, x4, dinv, W1, b1, W2, npad, blk):
    def body(a0_ref, a1_ref, x_ref, dinv_ref, w1_ref, b1_ref, w2_ref,
             g_ref, hs_ref):
        dv = dinv_ref[...]
        pre = jnp.broadcast_to(b1_ref[...][None, :], (dv.shape[0], 16))
        for f in range(3):
            p1f = dv * (a0_ref[f, :] + a1_ref[f, :]) + dv * dv * x_ref[:, f]
            pre = pre + p1f[:, None] * w1_ref[f, :][None, :]
        h1 = jnp.maximum(pre, 0.0)
        g = jnp.dot(h1, w2_ref[...], preferred_element_type=jnp.float32)
        g_ref[...] = g
        for f in range(7):
            hs_ref[f, :] = g[:, f] * dv

    return pl.pallas_call(
        body,
        grid=(npad // blk,),
        in_specs=[
            pl.BlockSpec((3, blk), lambda i: (0, i)),
            pl.BlockSpec((3, blk), lambda i: (0, i)),
            pl.BlockSpec((blk, 4), lambda i: (i, 0)),
            pl.BlockSpec((blk,), lambda i: (i,)),
            pl.BlockSpec((3, 16), lambda i: (0, 0)),
            pl.BlockSpec((16,), lambda i: (0,)),
            pl.BlockSpec((16, 8), lambda i: (0, 0)),
        ],
        out_specs=[
            pl.BlockSpec((blk, 8), lambda i: (i, 0)),
            pl.BlockSpec((7, blk), lambda i: (0, i)),
        ],
        out_shape=[
            jax.ShapeDtypeStruct((npad, 8), jnp.float32),
            jax.ShapeDtypeStruct((7, npad), jnp.float32),
        ],
    )(a0, a1, x4, dinv, W1, b1, W2)


def _tc_stage3(a0, a1, g8, dinv, b2p, npad, blk):
    def body(a0_ref, a1_ref, g8_ref, dinv_ref, b2_ref, out_ref):
        dv = dinv_ref[...]
        cols = []
        for f in range(7):
            cols.append((dv * (a0_ref[f, :] + a1_ref[f, :])
                         + dv * dv * g8_ref[:, f] + b2_ref[f])[:, None])
        o = jnp.concatenate(cols, axis=1)
        m = jnp.max(o, axis=1, keepdims=True)
        e = jnp.exp(o - m)
        ssum = jnp.sum(e, axis=1, keepdims=True)
        out_ref[...] = (o - m) - jnp.log(ssum)

    return pl.pallas_call(
        body,
        grid=(npad // blk,),
        in_specs=[
            pl.BlockSpec((7, blk), lambda i: (0, i)),
            pl.BlockSpec((7, blk), lambda i: (0, i)),
            pl.BlockSpec((blk, 8), lambda i: (i, 0)),
            pl.BlockSpec((blk,), lambda i: (i,)),
            pl.BlockSpec((8,), lambda i: (0,)),
        ],
        out_specs=pl.BlockSpec((blk, 7), lambda i: (i, 0)),
        out_shape=jax.ShapeDtypeStruct((npad, 7), jnp.float32),
    )(a0, a1, g8, dinv, b2p)


def kernel(x, edge_index, W1, b1, W2, b2):
    n = x.shape[0]
    e = edge_index.shape[1]
    blk = 2048
    npad = -(-(n + 1) // blk) * blk
    dummy = n
    chunk = 8 * _B
    groups = -(-e // (_NW * chunk))
    ept = groups * chunk
    epad = _NW * ept
    rows_per_tile = ept // _B

    ei = edge_index.astype(jnp.int32)
    pad = dummy + (jnp.arange(epad - e, dtype=jnp.int32) % (npad - n))
    src2d = jnp.concatenate([ei[0], pad]).reshape(epad // _B, _B)
    dst2d = jnp.concatenate([ei[1], pad]).reshape(epad // _B, _B)

    x4 = jnp.zeros((npad, 4), jnp.float32).at[:n, :3].set(x)
    zeros1 = jnp.zeros((npad,), jnp.float32)
    b2p = jnp.zeros((8,), jnp.float32).at[:7].set(b2)
    w2p = jnp.zeros((16, 8), jnp.float32).at[:, :7].set(W2)

    degp = _sc_degree(dst2d, zeros1, npad, rows_per_tile, 8)
    dinv, xs_t = _tc_stage1(degp[0], degp[1], x4, npad, blk, n)
    aggp = _sc_propagate(src2d, dst2d, xs_t, zeros1, npad, rows_per_tile, 3, 8)
    g8, hs_t = _tc_stage2(aggp[0], aggp[1], x4, dinv, W1, b1, w2p, npad, blk)
    agg2p = _sc_propagate(src2d, dst2d, hs_t, zeros1, npad, rows_per_tile, 7, 4)
    out = _tc_stage3(agg2p[0], agg2p[1], g8, dinv, b2p, npad, blk)
    return out[:n]


# speedup vs baseline: 51.6949x; 1.0617x over previous
import functools

import jax
import jax.numpy as jnp
from jax import lax
from jax.experimental import pallas as pl
from jax.experimental.pallas import tpu as pltpu
from jax.experimental.pallas import tpu_sc as plsc

_NC = 2
_NS = 16
_NW = _NC * _NS
_L = 16
_B = 128


def _mesh():
    return plsc.VectorSubcoreMesh(core_axis_name="c", subcore_axis_name="s")

_CP = pltpu.CompilerParams(use_tc_tiling_on_sc=False)


def _sc_degree(dst2d, zeros1, npad, rows_per_tile, sub):
    slab = npad // _NS

    @functools.partial(
        pl.kernel,
        out_type=jax.ShapeDtypeStruct((_NC, npad), jnp.float32),
        mesh=_mesh(),
        compiler_params=_CP,
        scratch_types=[
            pltpu.VMEM((sub, _B), jnp.int32),
            pltpu.VMEM((_B,), jnp.float32),
            pltpu.VMEM_SHARED((npad,), jnp.float32),
            pltpu.SemaphoreType.DMA,
        ],
    )
    def deg_kernel(dst_hbm, z_hbm, out_hbm, idx_v, ones_v, acc_sh, sem):
        c = lax.axis_index("c")
        s = lax.axis_index("s")
        rs = pl.ds(s * slab, slab)
        for i in range(_B // _L):
            ones_v[pl.ds(i * _L, _L)] = jnp.ones((_L,), jnp.float32)
        pltpu.sync_copy(z_hbm.at[rs], acc_sh.at[rs])
        plsc.subcore_barrier()
        base_row = (c * _NS + s) * rows_per_tile

        def body(g, carry):
            r0 = base_row + g * sub
            pltpu.sync_copy(dst_hbm.at[pl.ds(r0, sub)], idx_v)
            cps = [
                pltpu.async_copy(ones_v, acc_sh.at[idx_v.at[j]], sem, add=True)
                for j in range(sub)
            ]
            for cp in cps:
                cp.wait()
            return carry

        lax.fori_loop(0, rows_per_tile // sub, body, 0)
        plsc.subcore_barrier()
        pltpu.sync_copy(acc_sh.at[rs], out_hbm.at[c, rs])

    return deg_kernel(dst2d, zeros1)


def _sc_propagate(src2d, dst2d, table, zeros1, npad, rows_per_tile, nf, sub):
    slab = npad // _NS

    @functools.partial(
        pl.kernel,
        out_type=jax.ShapeDtypeStruct((_NC, nf, npad), jnp.float32),
        mesh=_mesh(),
        compiler_params=_CP,
        scratch_types=[
            pltpu.VMEM((sub, _B), jnp.int32),
            pltpu.VMEM((sub, _B), jnp.int32),
            pltpu.VMEM((sub * nf, _B), jnp.float32),
        ] + [pltpu.VMEM_SHARED((npad,), jnp.float32) for _ in range(2 * nf)] + [
            pltpu.SemaphoreType.DMA,
            pltpu.SemaphoreType.DMA,
        ],
    )
    def prop_kernel(src_hbm, dst_hbm, tab_hbm, z_hbm, out_hbm,
                    sidx_v, didx_v, rows_v, *rest):
        tab_sh = rest[:nf]
        acc_sh = rest[nf:2 * nf]
        gsem, ssem = rest[2 * nf], rest[2 * nf + 1]
        c = lax.axis_index("c")
        s = lax.axis_index("s")
        rs = pl.ds(s * slab, slab)
        for f in range(nf):
            pltpu.sync_copy(tab_hbm.at[f, rs], tab_sh[f].at[rs])
            pltpu.sync_copy(z_hbm.at[rs], acc_sh[f].at[rs])
        plsc.subcore_barrier()
        base_row = (c * _NS + s) * rows_per_tile

        def body(g, carry):
            r0 = base_row + g * sub
            pltpu.sync_copy(src_hbm.at[pl.ds(r0, sub)], sidx_v)
            pltpu.sync_copy(dst_hbm.at[pl.ds(r0, sub)], didx_v)
            gcs = [
                pltpu.async_copy(tab_sh[f].at[sidx_v.at[j]],
                                 rows_v.at[j * nf + f], gsem)
                for j in range(sub) for f in range(nf)
            ]
            scs = []
            for j in range(sub):
                for f in range(nf):
                    gcs[j * nf + f].wait()
                scs.extend(
                    pltpu.async_copy(rows_v.at[j * nf + f],
                                     acc_sh[f].at[didx_v.at[j]], ssem, add=True)
                    for f in range(nf)
                )
            for cp in scs:
                cp.wait()
            return carry

        lax.fori_loop(0, rows_per_tile // sub, body, 0)
        plsc.subcore_barrier()
        for f in range(nf):
            pltpu.sync_copy(acc_sh[f].at[rs], out_hbm.at[c, f, rs])

    return prop_kernel(src2d, dst2d, table, zeros1)


def _tc_stage1(deg0, deg1, x4, npad, blk, nreal):
    def body(d0_ref, d1_ref, x_ref, dinv_ref, xs_ref):
        deg = d0_ref[...] + d1_ref[...] + 1.0
        row = pl.program_id(0) * blk + lax.iota(jnp.int32, blk)
        dinv = jnp.where(row < nreal, lax.rsqrt(deg), 0.0)
        dinv_ref[...] = dinv
        for f in range(3):
            xs_ref[f, :] = x_ref[:, f] * dinv

    return pl.pallas_call(
        body,
        grid=(npad // blk,),
        in_specs=[
            pl.BlockSpec((blk,), lambda i: (i,)),
            pl.BlockSpec((blk,), lambda i: (i,)),
            pl.BlockSpec((blk, 4), lambda i: (i, 0)),
        ],
        out_specs=[
            pl.BlockSpec((blk,), lambda i: (i,)),
            pl.BlockSpec((3, blk), lambda i: (0, i)),
        ],
        out_shape=[
            jax.ShapeDtypeStruct((npad,), jnp.float32),
            jax.ShapeDtypeStruct((3, npad), jnp.float32),
        ],
    )(deg0, deg1, x4)


def _tc_stage2(a0, a1, x4, dinv, W1, b1, W2, npad, blk):
    def body(a0_ref, a1_ref, x_ref, dinv_ref, w1_ref, b1_ref, w2_ref,
             g_ref, hs_ref):
        dv = dinv_ref[...]
        pre = jnp.broadcast_to(b1_ref[...][None, :], (dv.shape[0], 16))
        for f in range(3):
            p1f = dv * (a0_ref[f, :] + a1_ref[f, :]) + dv * dv * x_ref[:, f]
            pre = pre + p1f[:, None] * w1_ref[f, :][None, :]
        h1 = jnp.maximum(pre, 0.0)
        g = jnp.dot(h1, w2_ref[...], preferred_element_type=jnp.float32)
        g_ref[...] = g
        for f in range(7):
            hs_ref[f, :] = g[:, f] * dv

    return pl.pallas_call(
        body,
        grid=(npad // blk,),
        in_specs=[
            pl.BlockSpec((3, blk), lambda i: (0, i)),
            pl.BlockSpec((3, blk), lambda i: (0, i)),
            pl.BlockSpec((blk, 4), lambda i: (i, 0)),
            pl.BlockSpec((blk,), lambda i: (i,)),
            pl.BlockSpec((3, 16), lambda i: (0, 0)),
            pl.BlockSpec((16,), lambda i: (0,)),
            pl.BlockSpec((16, 8), lambda i: (0, 0)),
        ],
        out_specs=[
            pl.BlockSpec((blk, 8), lambda i: (i, 0)),
            pl.BlockSpec((7, blk), lambda i: (0, i)),
        ],
        out_shape=[
            jax.ShapeDtypeStruct((npad, 8), jnp.float32),
            jax.ShapeDtypeStruct((7, npad), jnp.float32),
        ],
    )(a0, a1, x4, dinv, W1, b1, W2)


def _tc_stage3(a0, a1, g8, dinv, b2p, npad, blk):
    def body(a0_ref, a1_ref, g8_ref, dinv_ref, b2_ref, out_ref):
        dv = dinv_ref[...]
        cols = []
        for f in range(7):
            cols.append((dv * (a0_ref[f, :] + a1_ref[f, :])
                         + dv * dv * g8_ref[:, f] + b2_ref[f])[:, None])
        o = jnp.concatenate(cols, axis=1)
        m = jnp.max(o, axis=1, keepdims=True)
        e = jnp.exp(o - m)
        ssum = jnp.sum(e, axis=1, keepdims=True)
        out_ref[...] = (o - m) - jnp.log(ssum)

    return pl.pallas_call(
        body,
        grid=(npad // blk,),
        in_specs=[
            pl.BlockSpec((7, blk), lambda i: (0, i)),
            pl.BlockSpec((7, blk), lambda i: (0, i)),
            pl.BlockSpec((blk, 8), lambda i: (i, 0)),
            pl.BlockSpec((blk,), lambda i: (i,)),
            pl.BlockSpec((8,), lambda i: (0,)),
        ],
        out_specs=pl.BlockSpec((blk, 7), lambda i: (i, 0)),
        out_shape=jax.ShapeDtypeStruct((npad, 7), jnp.float32),
    )(a0, a1, g8, dinv, b2p)


def kernel(x, edge_index, W1, b1, W2, b2):
    n = x.shape[0]
    e = edge_index.shape[1]
    blk = 2048
    npad = -(-(n + 1) // blk) * blk
    dummy = n
    chunk = 8 * _B
    groups = -(-e // (_NW * chunk))
    ept = groups * chunk
    epad = _NW * ept
    rows_per_tile = ept // _B

    ei = edge_index.astype(jnp.int32)
    pad = dummy + (jnp.arange(epad - e, dtype=jnp.int32) % (npad - n))
    src2d = jnp.concatenate([ei[0], pad]).reshape(epad // _B, _B)
    dst2d = jnp.concatenate([ei[1], pad]).reshape(epad // _B, _B)

    x4 = jnp.zeros((npad, 4), jnp.float32).at[:n, :3].set(x)
    zeros1 = jnp.zeros((npad,), jnp.float32)
    b2p = jnp.zeros((8,), jnp.float32).at[:7].set(b2)
    w2p = jnp.zeros((16, 8), jnp.float32).at[:, :7].set(W2)

    degp = _sc_degree(dst2d, zeros1, npad, rows_per_tile, 16)
    dinv, xs_t = _tc_stage1(degp[0], degp[1], x4, npad, blk, n)
    aggp = _sc_propagate(src2d, dst2d, xs_t, zeros1, npad, rows_per_tile, 3, 16)
    g8, hs_t = _tc_stage2(aggp[0], aggp[1], x4, dinv, W1, b1, w2p, npad, blk)
    agg2p = _sc_propagate(src2d, dst2d, hs_t, zeros1, npad, rows_per_tile, 7, 8)
    out = _tc_stage3(agg2p[0], agg2p[1], g8, dinv, b2p, npad, blk)
    return out[:n]


# speedup vs baseline: 54.8074x; 1.0602x over previous
import functools

import jax
import jax.numpy as jnp
from jax import lax
from jax.experimental import pallas as pl
from jax.experimental.pallas import tpu as pltpu
from jax.experimental.pallas import tpu_sc as plsc

_NC = 2
_NS = 16
_NW = _NC * _NS
_L = 16
_B = 128


def _mesh():
    return plsc.VectorSubcoreMesh(core_axis_name="c", subcore_axis_name="s")

_CP = pltpu.CompilerParams(use_tc_tiling_on_sc=False)


def _sc_degree(dst2d, zeros1, npad, rows_per_tile, sub):
    slab = npad // _NS

    @functools.partial(
        pl.kernel,
        out_type=jax.ShapeDtypeStruct((_NC, npad), jnp.float32),
        mesh=_mesh(),
        compiler_params=_CP,
        scratch_types=[
            pltpu.VMEM((sub, _B), jnp.int32),
            pltpu.VMEM((_B,), jnp.float32),
            pltpu.VMEM_SHARED((npad,), jnp.float32),
            pltpu.SemaphoreType.DMA,
        ],
    )
    def deg_kernel(dst_hbm, z_hbm, out_hbm, idx_v, ones_v, acc_sh, sem):
        c = lax.axis_index("c")
        s = lax.axis_index("s")
        rs = pl.ds(s * slab, slab)
        for i in range(_B // _L):
            ones_v[pl.ds(i * _L, _L)] = jnp.ones((_L,), jnp.float32)
        pltpu.sync_copy(z_hbm.at[rs], acc_sh.at[rs])
        plsc.subcore_barrier()
        base_row = (c * _NS + s) * rows_per_tile

        def body(g, carry):
            r0 = base_row + g * sub
            pltpu.sync_copy(dst_hbm.at[pl.ds(r0, sub)], idx_v)
            cps = [
                pltpu.async_copy(ones_v, acc_sh.at[idx_v.at[j]], sem, add=True)
                for j in range(sub)
            ]
            for cp in cps:
                cp.wait()
            return carry

        lax.fori_loop(0, rows_per_tile // sub, body, 0)
        plsc.subcore_barrier()
        pltpu.sync_copy(acc_sh.at[rs], out_hbm.at[c, rs])

    return deg_kernel(dst2d, zeros1)


def _sc_propagate(src2d2, dst2d, tabs, zeros1, npad, rows_per_tile, nf, sub):
    slab = npad // _NS
    ntab = len(tabs)
    nf_per = [4] * (ntab - 1) + [nf - 4 * (ntab - 1)]

    @functools.partial(
        pl.kernel,
        out_type=jax.ShapeDtypeStruct((_NC, nf, npad), jnp.float32),
        mesh=_mesh(),
        compiler_params=pltpu.CompilerParams(use_tc_tiling_on_sc=False,
                                             needs_layout_passes=False),
        scratch_types=[
            pltpu.VMEM((sub, _B), jnp.int32),
            pltpu.VMEM((sub, _B), jnp.int32),
        ] + [pltpu.VMEM((sub, _B, 4), jnp.float32) for _ in range(ntab)] + [
            pltpu.VMEM((sub * nf, _B), jnp.float32),
        ] + [pltpu.VMEM_SHARED((npad,), jnp.float32) for _ in range(nf)] + [
            pltpu.SemaphoreType.DMA,
            pltpu.SemaphoreType.DMA,
        ],
    )
    def prop_kernel(src_hbm, dst_hbm, *args):
        tabs_hbm = args[:ntab]
        z_hbm = args[ntab]
        out_hbm = args[ntab + 1]
        sidx_v, didx_v = args[ntab + 2], args[ntab + 3]
        rows_v = args[ntab + 4:2 * ntab + 4]
        fm_v = args[2 * ntab + 4]
        acc_sh = args[2 * ntab + 5:2 * ntab + 5 + nf]
        gsem, ssem = args[2 * ntab + 5 + nf], args[2 * ntab + 6 + nf]
        c = lax.axis_index("c")
        s = lax.axis_index("s")
        rs = pl.ds(s * slab, slab)
        for f in range(nf):
            pltpu.sync_copy(z_hbm.at[rs], acc_sh[f].at[rs])
        plsc.subcore_barrier()
        base_row = (c * _NS + s) * rows_per_tile

        def body(g, carry):
            r0 = base_row + g * sub
            pltpu.sync_copy(src_hbm.at[pl.ds(r0, sub)], sidx_v)
            pltpu.sync_copy(dst_hbm.at[pl.ds(r0, sub)], didx_v)
            gcs = [
                pltpu.async_copy(tabs_hbm[t].at[sidx_v.at[j]],
                                 rows_v[t].at[j], gsem)
                for j in range(sub) for t in range(ntab)
            ]
            scs = []
            for j in range(sub):
                for t in range(ntab):
                    gcs[j * ntab + t].wait()
                jv = jnp.full((_L,), j, jnp.int32)
                fidx = 0
                for t in range(ntab):
                    for fl in range(nf_per[t]):
                        for k in range(_B // _L):
                            q = (lax.iota(jnp.int32, _L) + k * _L) * 4 + fl
                            rv = lax.shift_right_logical(q, 3)
                            fv = lax.bitwise_and(q, 7)
                            vals = plsc.load_gather(rows_v[t], [jv, rv, fv])
                            fm_v[j * nf + fidx, pl.ds(k * _L, _L)] = vals
                        fidx += 1
                scs.extend(
                    pltpu.async_copy(fm_v.at[j * nf + f],
                                     acc_sh[f].at[didx_v.at[j]], ssem, add=True)
                    for f in range(nf)
                )
            for cp in scs:
                cp.wait()
            return carry

        lax.fori_loop(0, rows_per_tile // sub, body, 0)
        plsc.subcore_barrier()
        for f in range(nf):
            pltpu.sync_copy(acc_sh[f].at[rs], out_hbm.at[c, f, rs])

    return prop_kernel(src2d2, dst2d, *tabs, zeros1)


def _tc_stage1(deg0, deg1, x4, npad, blk, nreal):
    def body(d0_ref, d1_ref, x_ref, dinv_ref, xs_ref):
        deg = d0_ref[...] + d1_ref[...] + 1.0
        row = pl.program_id(0) * blk + lax.iota(jnp.int32, blk)
        dinv = jnp.where(row < nreal, lax.rsqrt(deg), 0.0)
        dinv_ref[...] = dinv
        xs_ref[...] = x_ref[...] * dinv[:, None]

    return pl.pallas_call(
        body,
        grid=(npad // blk,),
        in_specs=[
            pl.BlockSpec((blk,), lambda i: (i,)),
            pl.BlockSpec((blk,), lambda i: (i,)),
            pl.BlockSpec((blk, 4), lambda i: (i, 0)),
        ],
        out_specs=[
            pl.BlockSpec((blk,), lambda i: (i,)),
            pl.BlockSpec((blk, 4), lambda i: (i, 0)),
        ],
        out_shape=[
            jax.ShapeDtypeStruct((npad,), jnp.float32),
            jax.ShapeDtypeStruct((npad, 4), jnp.float32),
        ],
    )(deg0, deg1, x4)


def _tc_stage2(a0, a1, x4, dinv, W1, b1, W2, npad, blk):
    def body(a0_ref, a1_ref, x_ref, dinv_ref, w1_ref, b1_ref, w2_ref,
             g_ref, hsa_ref, hsb_ref):
        dv = dinv_ref[...]
        pre = jnp.broadcast_to(b1_ref[...][None, :], (dv.shape[0], 16))
        for f in range(3):
            p1f = dv * (a0_ref[f, :] + a1_ref[f, :]) + dv * dv * x_ref[:, f]
            pre = pre + p1f[:, None] * w1_ref[f, :][None, :]
        h1 = jnp.maximum(pre, 0.0)
        g = jnp.dot(h1, w2_ref[...], preferred_element_type=jnp.float32)
        g_ref[...] = g
        hs = g * dv[:, None]
        hsa_ref[...] = hs[:, :4]
        hsb_ref[...] = hs[:, 4:8]

    return pl.pallas_call(
        body,
        grid=(npad // blk,),
        in_specs=[
            pl.BlockSpec((3, blk), lambda i: (0, i)),
            pl.BlockSpec((3, blk), lambda i: (0, i)),
            pl.BlockSpec((blk, 4), lambda i: (i, 0)),
            pl.BlockSpec((blk,), lambda i: (i,)),
            pl.BlockSpec((3, 16), lambda i: (0, 0)),
            pl.BlockSpec((16,), lambda i: (0,)),
            pl.BlockSpec((16, 8), lambda i: (0, 0)),
        ],
        out_specs=[
            pl.BlockSpec((blk, 8), lambda i: (i, 0)),
            pl.BlockSpec((blk, 4), lambda i: (i, 0)),
            pl.BlockSpec((blk, 4), lambda i: (i, 0)),
        ],
        out_shape=[
            jax.ShapeDtypeStruct((npad, 8), jnp.float32),
            jax.ShapeDtypeStruct((npad, 4), jnp.float32),
            jax.ShapeDtypeStruct((npad, 4), jnp.float32),
        ],
    )(a0, a1, x4, dinv, W1, b1, W2)


def _tc_stage3(a0, a1, g8, dinv, b2p, npad, blk):
    def body(a0_ref, a1_ref, g8_ref, dinv_ref, b2_ref, out_ref):
        dv = dinv_ref[...]
        cols = []
        for f in range(7):
            cols.append((dv * (a0_ref[f, :] + a1_ref[f, :])
                         + dv * dv * g8_ref[:, f] + b2_ref[f])[:, None])
        o = jnp.concatenate(cols, axis=1)
        m = jnp.max(o, axis=1, keepdims=True)
        e = jnp.exp(o - m)
        ssum = jnp.sum(e, axis=1, keepdims=True)
        out_ref[...] = (o - m) - jnp.log(ssum)

    return pl.pallas_call(
        body,
        grid=(npad // blk,),
        in_specs=[
            pl.BlockSpec((7, blk), lambda i: (0, i)),
            pl.BlockSpec((7, blk), lambda i: (0, i)),
            pl.BlockSpec((blk, 8), lambda i: (i, 0)),
            pl.BlockSpec((blk,), lambda i: (i,)),
            pl.BlockSpec((8,), lambda i: (0,)),
        ],
        out_specs=pl.BlockSpec((blk, 7), lambda i: (i, 0)),
        out_shape=jax.ShapeDtypeStruct((npad, 7), jnp.float32),
    )(a0, a1, g8, dinv, b2p)


def kernel(x, edge_index, W1, b1, W2, b2):
    n = x.shape[0]
    e = edge_index.shape[1]
    blk = 2048
    npad = -(-(n + 1) // blk) * blk
    dummy = n
    chunk = 8 * _B
    groups = -(-e // (_NW * chunk))
    ept = groups * chunk
    epad = _NW * ept
    rows_per_tile = ept // _B

    ei = edge_index.astype(jnp.int32)
    pad = dummy + (jnp.arange(epad - e, dtype=jnp.int32) % (npad - n))
    src2d2 = (jnp.concatenate([ei[0], pad]) * 2).reshape(epad // _B, _B)
    dst2d = jnp.concatenate([ei[1], pad]).reshape(epad // _B, _B)

    x4 = jnp.zeros((npad, 4), jnp.float32).at[:n, :3].set(x)
    zeros1 = jnp.zeros((npad,), jnp.float32)
    b2p = jnp.zeros((8,), jnp.float32).at[:7].set(b2)
    w2p = jnp.zeros((16, 8), jnp.float32).at[:, :7].set(W2)

    degp = _sc_degree(dst2d, zeros1, npad, rows_per_tile, 16)
    dinv, xs4 = _tc_stage1(degp[0], degp[1], x4, npad, blk, n)
    aggp = _sc_propagate(src2d2, dst2d, [xs4], zeros1, npad,
                         rows_per_tile, 3, 16)
    g8, hsa, hsb = _tc_stage2(aggp[0], aggp[1], x4, dinv, W1, b1, w2p,
                              npad, blk)
    agg2p = _sc_propagate(src2d2, dst2d, [hsa, hsb], zeros1, npad,
                          rows_per_tile, 7, 8)
    out = _tc_stage3(agg2p[0], agg2p[1], g8, dinv, b2p, npad, blk)
    return out[:n]


# speedup vs baseline: 57.3143x; 1.0457x over previous
import functools

import jax
import jax.numpy as jnp
from jax import lax
from jax.experimental import pallas as pl
from jax.experimental.pallas import tpu as pltpu
from jax.experimental.pallas import tpu_sc as plsc

_NC = 2
_NS = 16
_NW = _NC * _NS
_L = 16
_B = 128


def _mesh():
    return plsc.VectorSubcoreMesh(core_axis_name="c", subcore_axis_name="s")

_CP = pltpu.CompilerParams(use_tc_tiling_on_sc=False)


def _sc_degree(dst2d, zeros1, npad, rows_per_tile, sub):
    slab = npad // _NS

    @functools.partial(
        pl.kernel,
        out_type=jax.ShapeDtypeStruct((_NC, npad), jnp.float32),
        mesh=_mesh(),
        compiler_params=_CP,
        scratch_types=[
            pltpu.VMEM((sub, _B), jnp.int32),
            pltpu.VMEM((_B,), jnp.float32),
            pltpu.VMEM_SHARED((npad,), jnp.float32),
            pltpu.SemaphoreType.DMA,
        ],
    )
    def deg_kernel(dst_hbm, z_hbm, out_hbm, idx_v, ones_v, acc_sh, sem):
        c = lax.axis_index("c")
        s = lax.axis_index("s")
        rs = pl.ds(s * slab, slab)
        for i in range(_B // _L):
            ones_v[pl.ds(i * _L, _L)] = jnp.ones((_L,), jnp.float32)
        pltpu.sync_copy(z_hbm.at[rs], acc_sh.at[rs])
        plsc.subcore_barrier()
        base_row = (c * _NS + s) * rows_per_tile

        def body(g, carry):
            r0 = base_row + g * sub
            pltpu.sync_copy(dst_hbm.at[pl.ds(r0, sub)], idx_v)
            cps = [
                pltpu.async_copy(ones_v, acc_sh.at[idx_v.at[j]], sem, add=True)
                for j in range(sub)
            ]
            for cp in cps:
                cp.wait()
            return carry

        lax.fori_loop(0, rows_per_tile // sub, body, 0)
        plsc.subcore_barrier()
        pltpu.sync_copy(acc_sh.at[rs], out_hbm.at[c, rs])

    return deg_kernel(dst2d, zeros1)


def _sc_propagate(src2d2, dst2d, tabs, zeros1, npad, rows_per_tile, nf, sub):
    slab = npad // _NS
    ntab = len(tabs)
    nf_per = [4] * (ntab - 1) + [nf - 4 * (ntab - 1)]

    @functools.partial(
        pl.kernel,
        out_type=jax.ShapeDtypeStruct((_NC, nf, npad), jnp.float32),
        mesh=_mesh(),
        compiler_params=pltpu.CompilerParams(use_tc_tiling_on_sc=False,
                                             needs_layout_passes=False),
        scratch_types=[
            pltpu.VMEM((sub, _B), jnp.int32),
            pltpu.VMEM((sub, _B), jnp.int32),
        ] + [pltpu.VMEM((sub, _B, 4), jnp.float32) for _ in range(ntab)] + [
            pltpu.VMEM((sub * nf, _B), jnp.float32),
        ] + [pltpu.VMEM_SHARED((npad,), jnp.float32) for _ in range(nf)] + [
            pltpu.SemaphoreType.DMA,
            pltpu.SemaphoreType.DMA,
        ],
    )
    def prop_kernel(src_hbm, dst_hbm, *args):
        tabs_hbm = args[:ntab]
        z_hbm = args[ntab]
        out_hbm = args[ntab + 1]
        sidx_v, didx_v = args[ntab + 2], args[ntab + 3]
        rows_v = args[ntab + 4:2 * ntab + 4]
        fm_v = args[2 * ntab + 4]
        acc_sh = args[2 * ntab + 5:2 * ntab + 5 + nf]
        gsem, ssem = args[2 * ntab + 5 + nf], args[2 * ntab + 6 + nf]
        c = lax.axis_index("c")
        s = lax.axis_index("s")
        rs = pl.ds(s * slab, slab)
        for f in range(nf):
            pltpu.sync_copy(z_hbm.at[rs], acc_sh[f].at[rs])
        plsc.subcore_barrier()
        base_row = (c * _NS + s) * rows_per_tile

        def body(g, carry):
            r0 = base_row + g * sub
            pltpu.sync_copy(src_hbm.at[pl.ds(r0, sub)], sidx_v)
            pltpu.sync_copy(dst_hbm.at[pl.ds(r0, sub)], didx_v)
            gcs = [
                pltpu.async_copy(tabs_hbm[t].at[sidx_v.at[j]],
                                 rows_v[t].at[j], gsem)
                for j in range(sub) for t in range(ntab)
            ]
            scs = []
            for j in range(sub):
                for t in range(ntab):
                    gcs[j * ntab + t].wait()
                jv = jnp.full((_L,), j, jnp.int32)
                fidx = 0
                for t in range(ntab):
                    for fl in range(nf_per[t]):
                        for k in range(_B // _L):
                            q = (lax.iota(jnp.int32, _L) + k * _L) * 4 + fl
                            rv = lax.shift_right_logical(q, 3)
                            fv = lax.bitwise_and(q, 7)
                            vals = plsc.load_gather(rows_v[t], [jv, rv, fv])
                            fm_v[j * nf + fidx, pl.ds(k * _L, _L)] = vals
                        fidx += 1
                scs.extend(
                    pltpu.async_copy(fm_v.at[j * nf + f],
                                     acc_sh[f].at[didx_v.at[j]], ssem, add=True)
                    for f in range(nf)
                )
            for cp in scs:
                cp.wait()
            return carry

        lax.fori_loop(0, rows_per_tile // sub, body, 0)
        plsc.subcore_barrier()
        for f in range(nf):
            pltpu.sync_copy(acc_sh[f].at[rs], out_hbm.at[c, f, rs])

    return prop_kernel(src2d2, dst2d, *tabs, zeros1)


def _tc_stage1(deg0, deg1, x4, npad, blk, nreal):
    def body(d0_ref, d1_ref, x_ref, dinv_ref, xs_ref):
        deg = d0_ref[...] + d1_ref[...] + 1.0
        row = pl.program_id(0) * blk + lax.iota(jnp.int32, blk)
        dinv = jnp.where(row < nreal, lax.rsqrt(deg), 0.0)
        dinv_ref[...] = dinv
        xs_ref[...] = x_ref[...] * dinv[:, None]

    return pl.pallas_call(
        body,
        grid=(npad // blk,),
        in_specs=[
            pl.BlockSpec((blk,), lambda i: (i,)),
            pl.BlockSpec((blk,), lambda i: (i,)),
            pl.BlockSpec((blk, 4), lambda i: (i, 0)),
        ],
        out_specs=[
            pl.BlockSpec((blk,), lambda i: (i,)),
            pl.BlockSpec((blk, 4), lambda i: (i, 0)),
        ],
        out_shape=[
            jax.ShapeDtypeStruct((npad,), jnp.float32),
            jax.ShapeDtypeStruct((npad, 4), jnp.float32),
        ],
    )(deg0, deg1, x4)


def _tc_stage2(a0, a1, x4, dinv, W1, b1, W2, npad, blk):
    def body(a0_ref, a1_ref, x_ref, dinv_ref, w1_ref, b1_ref, w2_ref,
             g_ref, hsa_ref, hsb_ref):
        dv = dinv_ref[...]
        pre = jnp.broadcast_to(b1_ref[...][None, :], (dv.shape[0], 16))
        for f in range(3):
            p1f = dv * (a0_ref[f, :] + a1_ref[f, :]) + dv * dv * x_ref[:, f]
            pre = pre + p1f[:, None] * w1_ref[f, :][None, :]
        h1 = jnp.maximum(pre, 0.0)
        g = jnp.dot(h1, w2_ref[...], preferred_element_type=jnp.float32)
        g_ref[...] = g
        hs = g * dv[:, None]
        hsa_ref[...] = hs[:, :4]
        hsb_ref[...] = hs[:, 4:8]

    return pl.pallas_call(
        body,
        grid=(npad // blk,),
        in_specs=[
            pl.BlockSpec((3, blk), lambda i: (0, i)),
            pl.BlockSpec((3, blk), lambda i: (0, i)),
            pl.BlockSpec((blk, 4), lambda i: (i, 0)),
            pl.BlockSpec((blk,), lambda i: (i,)),
            pl.BlockSpec((3, 16), lambda i: (0, 0)),
            pl.BlockSpec((16,), lambda i: (0,)),
            pl.BlockSpec((16, 8), lambda i: (0, 0)),
        ],
        out_specs=[
            pl.BlockSpec((blk, 8), lambda i: (i, 0)),
            pl.BlockSpec((blk, 4), lambda i: (i, 0)),
            pl.BlockSpec((blk, 4), lambda i: (i, 0)),
        ],
        out_shape=[
            jax.ShapeDtypeStruct((npad, 8), jnp.float32),
            jax.ShapeDtypeStruct((npad, 4), jnp.float32),
            jax.ShapeDtypeStruct((npad, 4), jnp.float32),
        ],
    )(a0, a1, x4, dinv, W1, b1, W2)


def _tc_stage3(a0, a1, g8, dinv, b2p, npad, blk):
    def body(a0_ref, a1_ref, g8_ref, dinv_ref, b2_ref, out_ref):
        dv = dinv_ref[...]
        cols = []
        for f in range(7):
            cols.append((dv * (a0_ref[f, :] + a1_ref[f, :])
                         + dv * dv * g8_ref[:, f] + b2_ref[f])[:, None])
        o = jnp.concatenate(cols, axis=1)
        m = jnp.max(o, axis=1, keepdims=True)
        e = jnp.exp(o - m)
        ssum = jnp.sum(e, axis=1, keepdims=True)
        out_ref[...] = (o - m) - jnp.log(ssum)

    return pl.pallas_call(
        body,
        grid=(npad // blk,),
        in_specs=[
            pl.BlockSpec((7, blk), lambda i: (0, i)),
            pl.BlockSpec((7, blk), lambda i: (0, i)),
            pl.BlockSpec((blk, 8), lambda i: (i, 0)),
            pl.BlockSpec((blk,), lambda i: (i,)),
            pl.BlockSpec((8,), lambda i: (0,)),
        ],
        out_specs=pl.BlockSpec((blk, 7), lambda i: (i, 0)),
        out_shape=jax.ShapeDtypeStruct((npad, 7), jnp.float32),
    )(a0, a1, g8, dinv, b2p)


def kernel(x, edge_index, W1, b1, W2, b2):
    n = x.shape[0]
    e = edge_index.shape[1]
    blk = 2048
    npad = -(-(n + 1) // blk) * blk
    dummy = n
    chunk = 8 * _B
    groups = -(-e // (_NW * chunk))
    ept = groups * chunk
    epad = _NW * ept
    rows_per_tile = ept // _B

    ei = edge_index.astype(jnp.int32)
    pad = dummy + (jnp.arange(epad - e, dtype=jnp.int32) % (npad - n))
    src2d2 = (jnp.concatenate([ei[0], pad]) * 2).reshape(epad // _B, _B)
    dst2d = jnp.concatenate([ei[1], pad]).reshape(epad // _B, _B)

    x4 = jnp.zeros((npad, 4), jnp.float32).at[:n, :3].set(x)
    zeros1 = jnp.zeros((npad,), jnp.float32)
    b2p = jnp.zeros((8,), jnp.float32).at[:7].set(b2)
    w2p = jnp.zeros((16, 8), jnp.float32).at[:, :7].set(W2)

    degp = _sc_degree(dst2d, zeros1, npad, rows_per_tile, 16)
    dinv, xs4 = _tc_stage1(degp[0], degp[1], x4, npad, blk, n)
    aggp = _sc_propagate(src2d2, dst2d, [xs4], zeros1, npad,
                         rows_per_tile, 3, 16)
    g8, hsa, hsb = _tc_stage2(aggp[0], aggp[1], x4, dinv, W1, b1, w2p,
                              npad, blk)
    agg2p = _sc_propagate(src2d2, dst2d, [hsa, hsb], zeros1, npad,
                          rows_per_tile, 7, 16)
    out = _tc_stage3(agg2p[0], agg2p[1], g8, dinv, b2p, npad, blk)
    return out[:n]
